# baseline (speedup 1.0000x reference)
import functools

import jax
import jax.numpy as jnp
from jax import lax
from jax.experimental import pallas as pl
from jax.experimental.pallas import tpu as pltpu
from jax.experimental.pallas import tpu_sc as plsc

N = 100000
E = 6400000
NPAD = 100096
ROWS = NPAD // 128
NT = 32
EPT = E // NT
CHUNK = 2000
NCHUNK = EPT // CHUNK
SLICE = NPAD // 16

_MESH = dict(core_axis_name="c", subcore_axis_name="s")


def _worker(c, s):
    return s * 2 + c


def _fill(ref, n, value):
    vec = jnp.full((16,), value, jnp.float32)

    def _body(i, carry):
        ref[pl.ds(i * 16, 16)] = vec
        return carry

    lax.fori_loop(0, n // 16, _body, None)


SA = 280000
SB = 120000
HALF = E // 2
assert 8 * (SA + SB) == HALF and SA % (2 * CHUNK) == 0 and SB % (2 * CHUNK) == 0


@functools.partial(
    pl.kernel,
    out_type=jax.ShapeDtypeStruct((18 * NPAD,), jnp.float32),
    mesh=plsc.VectorSubcoreMesh(**_MESH),
    scratch_types=[
        pltpu.VMEM((CHUNK,), jnp.int32),
        pltpu.VMEM((CHUNK,), jnp.int32),
        pltpu.VMEM((CHUNK,), jnp.float32),
        pltpu.VMEM((SLICE,), jnp.float32),
        pltpu.VMEM((NPAD,), jnp.float32),
        pltpu.VMEM_SHARED((NPAD,), jnp.float32),
        pltpu.SemaphoreType.DMA,
        pltpu.SemaphoreType.DMA,
        pltpu.SemaphoreType.DMA,
        pltpu.SemaphoreType.DMA,
    ],
    compiler_params=pltpu.CompilerParams(needs_layout_passes=False),
)
def _sc_histogram(col_hbm, out_hbm, col0, col1, ones_v, stage_v, acc_v, acc,
                  sl0, sl1, ss0, ss1):
    c = lax.axis_index("c")
    s = lax.axis_index("s")

    _fill(ones_v, CHUNK, 1.0)
    _fill(stage_v, SLICE, 0.0)
    sl = pl.ds(s * SLICE, SLICE)
    pltpu.sync_copy(stage_v, acc.at[sl])
    plsc.subcore_barrier()

    bufs = ((col0, sl0, ss0), (col1, sl1, ss1))
    ones16 = jnp.ones((16,), jnp.float32)

    def _src(base, ci):
        return col_hbm.at[pl.ds(base + ci * CHUNK, CHUNK)]

    def _ls(base, ci, b):
        pltpu.async_copy(_src(base, ci), bufs[b][0], bufs[b][1])

    def _lw(base, ci, b):
        pltpu.make_async_copy(_src(base, ci), bufs[b][0], bufs[b][1]).wait()

    @pl.when(s < 8)
    def _stream_half():
        base = c * HALF + s * SA
        nchunk = SA // CHUNK

        def _scs(b):
            pltpu.async_copy(ones_v, acc.at[bufs[b][0]], bufs[b][2],
                             add=True)

        def _scw(b):
            pltpu.make_async_copy(ones_v, acc.at[bufs[b][0]],
                                  bufs[b][2]).wait()

        _ls(base, 0, 0)
        _ls(base, 1, 1)

        def _body(k, carry):
            c0 = 2 * k
            _lw(base, c0, 0)
            _scs(0)
            _lw(base, c0 + 1, 1)
            _scs(1)
            _scw(0)
            _ls(base, c0 + 2, 0)
            _scw(1)
            _ls(base, c0 + 3, 1)
            return carry

        lax.fori_loop(0, nchunk // 2 - 1, _body, None)
        _lw(base, nchunk - 2, 0)
        _scs(0)
        _lw(base, nchunk - 1, 1)
        _scs(1)
        _scw(0)
        _scw(1)

    @pl.when(s >= 8)
    def _local_half():
        base = c * HALF + 8 * SA + (s - 8) * SB
        nchunk = SB // CHUNK
        _fill(acc_v, NPAD, 0.0)

        def _count(b):
            cb = bufs[b][0]

            def _cbody(j, carry):
                idx = cb[pl.ds(j * 16, 16)]
                plsc.addupdate_scatter(acc_v, [idx], ones16)
                return carry

            lax.fori_loop(0, CHUNK // 16, _cbody, None)

        _ls(base, 0, 0)
        _ls(base, 1, 1)

        def _body(k, carry):
            c0 = 2 * k
            _lw(base, c0, 0)
            _count(0)
            _ls(base, c0 + 2, 0)
            _lw(base, c0 + 1, 1)
            _count(1)
            _ls(base, c0 + 3, 1)
            return carry

        lax.fori_loop(0, nchunk // 2 - 1, _body, None)
        _lw(base, nchunk - 2, 0)
        _count(0)
        _lw(base, nchunk - 1, 1)
        _count(1)

    plsc.subcore_barrier()
    pltpu.sync_copy(acc.at[sl], stage_v)
    pltpu.sync_copy(stage_v, out_hbm.at[pl.ds(c * NPAD + s * SLICE, SLICE)])

    @pl.when(s >= 8)
    def _local_out():
        pltpu.sync_copy(
            acc_v,
            out_hbm.at[pl.ds((2 + c * 8 + (s - 8)) * NPAD, NPAD)])


def _make_edge_pass(packed, chunk):
    nchan = 2 if packed else 1
    nchunk = EPT // chunk
    assert EPT % chunk == 0 and nchunk % 2 == 0 and chunk % 16 == 0
    tab_dt = jnp.int32 if packed else jnp.float32
    out_sds = jax.ShapeDtypeStruct((nchan * 2 * NPAD,), jnp.float32)
    per_buf = [
        pltpu.VMEM((chunk,), jnp.int32),
        pltpu.VMEM((chunk,), jnp.int32),
    ] + [pltpu.VMEM((chunk,), jnp.float32)] * nchan + [
        pltpu.SemaphoreType.DMA,
        pltpu.SemaphoreType.DMA,
    ]
    scratch = per_buf + per_buf + [
        pltpu.VMEM((NPAD,), tab_dt),
    ] + [pltpu.VMEM_SHARED((NPAD,), jnp.float32)] * nchan

    pieces = []
    _off = 0
    while _off < SLICE:
        _sz = min(chunk, SLICE - _off)
        pieces.append((_off, _sz))
        _off += _sz
    assert all(sz > 0 and off % 8 == 0 and sz % 8 == 0 for off, sz in pieces)

    @functools.partial(
        pl.kernel,
        out_type=out_sds,
        mesh=plsc.VectorSubcoreMesh(**_MESH),
        scratch_types=scratch,
        compiler_params=pltpu.CompilerParams(needs_layout_passes=False),
    )
    def _pass(*args):
        tab_hbm, row_hbm, col_hbm, out_hbm = args[:4]
        rest = args[4:]
        nb = len(per_buf)
        bufs = (rest[:nb], rest[nb:2 * nb])
        table_v = rest[2 * nb]
        accs = rest[2 * nb + 1:]
        stage_v = bufs[0][2]

        c = lax.axis_index("c")
        s = lax.axis_index("s")

        pltpu.sync_copy(tab_hbm, table_v)
        _fill(stage_v, chunk, 0.0)
        for acc in accs:
            for off, sz in pieces:
                pltpu.sync_copy(stage_v.at[pl.ds(0, sz)],
                                acc.at[pl.ds(s * SLICE + off, sz)])
        plsc.subcore_barrier()

        base = _worker(c, s) * EPT

        def _rsrc(ci):
            return row_hbm.at[pl.ds(base + ci * chunk, chunk)]

        def _csrc(ci):
            return col_hbm.at[pl.ds(base + ci * chunk, chunk)]

        def _ls(ci, b):
            bb = bufs[b]
            pltpu.async_copy(_rsrc(ci), bb[0], bb[nchan + 2])
            pltpu.async_copy(_csrc(ci), bb[1], bb[nchan + 2])

        def _lw(ci, b):
            bb = bufs[b]
            pltpu.make_async_copy(_rsrc(ci), bb[0], bb[nchan + 2]).wait()
            pltpu.make_async_copy(_csrc(ci), bb[1], bb[nchan + 2]).wait()

        mask_hi = jnp.full((16,), -65536, jnp.int32)

        def _gather(b):
            bb = bufs[b]

            def _gbody(j, carry):
                dj = pl.ds(j * 16, 16)
                idx = bb[0][dj]
                w = plsc.load_gather(table_v, [idx])
                if packed:
                    bb[2][dj] = plsc.bitcast(
                        lax.shift_left(w, jnp.full((16,), 16, jnp.int32)),
                        jnp.float32)
                    bb[3][dj] = plsc.bitcast(lax.bitwise_and(w, mask_hi),
                                             jnp.float32)
                else:
                    bb[2][dj] = w
                return carry

            lax.fori_loop(0, chunk // 16, _gbody, None)

        def _scs(b):
            bb = bufs[b]
            for k in range(nchan):
                pltpu.async_copy(bb[2 + k], accs[k].at[bb[1]], bb[nchan + 3],
                                 add=True)

        def _scw(b):
            bb = bufs[b]
            for k in range(nchan):
                pltpu.make_async_copy(bb[2 + k], accs[k].at[bb[1]],
                                      bb[nchan + 3]).wait()

        _ls(0, 0)
        _ls(1, 1)

        def _body(k, carry):
            c0 = 2 * k
            _lw(c0, 0)
            _gather(0)
            _scs(0)
            _lw(c0 + 1, 1)
            _gather(1)
            _scs(1)
            _scw(0)
            _ls(c0 + 2, 0)
            _scw(1)
            _ls(c0 + 3, 1)
            return carry

        lax.fori_loop(0, nchunk // 2 - 1, _body, None)

        _lw(nchunk - 2, 0)
        _gather(0)
        _scs(0)
        _lw(nchunk - 1, 1)
        _gather(1)
        _scs(1)
        _scw(0)
        _scw(1)

        plsc.subcore_barrier()
        for k, acc in enumerate(accs):
            for off, sz in pieces:
                pltpu.sync_copy(acc.at[pl.ds(s * SLICE + off, sz)],
                                stage_v.at[pl.ds(0, sz)])
                pltpu.sync_copy(
                    stage_v.at[pl.ds(0, sz)],
                    out_hbm.at[pl.ds((2 * k + c) * NPAD + s * SLICE + off,
                                     sz)])

    return _pass


_sc_edge1 = _make_edge_pass(False, 4000)
_sc_edge2 = _make_edge_pass(True, 2000)


def _t1_body(cnt_ref, x_ref, dinv_ref, dis_ref, q_ref, xod_ref):
    deg = jnp.sum(cnt_ref[...], axis=0) + 1.0
    dis = lax.rsqrt(deg)
    dinv = 1.0 / deg
    x = x_ref[...]
    dinv_ref[...] = dinv
    dis_ref[...] = dis
    q_ref[...] = dis * x
    xod_ref[...] = x * dinv


def _t2_body(dis_ref, apart_ref, xod_ref, dinv_ref, w1_ref, b1_ref, w2_ref,
             rw_ref, pod0_ref, pod1_ref):
    dis = dis_ref[...]
    svec = dis * (apart_ref[0] + apart_ref[1]) + xod_ref[...]
    p0 = jnp.zeros_like(svec)
    p1 = jnp.zeros_like(svec)
    for k in range(16):
        hk = jnp.maximum(svec * w1_ref[0, k] + b1_ref[k], 0.0)
        p0 = p0 + hk * w2_ref[k, 0]
        p1 = p1 + hk * w2_ref[k, 1]
    dinv = dinv_ref[...]
    u0 = lax.bitcast_convert_type(
        lax.convert_element_type(dis * p0, jnp.bfloat16),
        jnp.uint16).astype(jnp.int32)
    u1 = lax.bitcast_convert_type(
        lax.convert_element_type(dis * p1, jnp.bfloat16),
        jnp.uint16).astype(jnp.int32)
    rw_ref[...] = jnp.bitwise_or(jnp.left_shift(u1, 16), u0)
    pod0_ref[...] = p0 * dinv
    pod1_ref[...] = p1 * dinv


def _t3_body(dis_ref, bc0_ref, bc1_ref, pod0_ref, pod1_ref, b2_ref,
             o0_ref, o1_ref):
    dis = dis_ref[...]
    t0 = dis * (bc0_ref[0] + bc0_ref[1]) + pod0_ref[...] + b2_ref[0]
    t1 = dis * (bc1_ref[0] + bc1_ref[1]) + pod1_ref[...] + b2_ref[1]
    m = jnp.maximum(t0, t1)
    e0 = jnp.exp(t0 - m)
    e1 = jnp.exp(t1 - m)
    den = e0 + e1
    o0_ref[...] = e0 / den
    o1_ref[...] = e1 / den


_f32 = jnp.float32
_blk = lambda: pl.BlockSpec(memory_space=pltpu.MemorySpace.VMEM)
_smem = lambda: pl.BlockSpec(memory_space=pltpu.MemorySpace.SMEM)
_VEC = jax.ShapeDtypeStruct((ROWS, 128), _f32)

_t1 = pl.pallas_call(
    _t1_body,
    out_shape=[_VEC, _VEC, _VEC, _VEC],
    in_specs=[_blk(), _blk()],
    out_specs=[_blk(), _blk(), _blk(), _blk()],
)

_VECI = jax.ShapeDtypeStruct((ROWS, 128), jnp.int32)
_t2 = pl.pallas_call(
    _t2_body,
    out_shape=[_VECI, _VEC, _VEC],
    in_specs=[_blk(), _blk(), _blk(), _blk(), _smem(), _smem(), _smem()],
    out_specs=[_blk(), _blk(), _blk()],
)

_t3 = pl.pallas_call(
    _t3_body,
    out_shape=[_VEC, _VEC],
    in_specs=[_blk(), _blk(), _blk(), _blk(), _blk(), _smem()],
    out_specs=[_blk(), _blk()],
)


def kernel(x, edge_index, W1, b1, W2, b2):
    row = edge_index[0]
    col = edge_index[1]
    xp = jnp.pad(x[:, 0], (0, NPAD - N)).reshape(ROWS, 128)

    cnt = _sc_histogram(col)
    dinv, dis, q, xod = _t1(cnt.reshape(18, ROWS, 128), xp)

    a_part = _sc_edge1(q.reshape(NPAD), row, col)
    rw, pod0, pod1 = _t2(dis, a_part.reshape(2, ROWS, 128), xod, dinv,
                         W1, b1, W2)

    b_part = _sc_edge2(rw.reshape(NPAD), row, col)
    b_part = b_part.reshape(2, 2, NPAD)
    bc0 = b_part[0].reshape(2, ROWS, 128)
    bc1 = b_part[1].reshape(2, ROWS, 128)
    o0, o1 = _t3(dis, bc0, bc1, pod0, pod1, b2)

    return jnp.stack([o0.reshape(NPAD)[:N], o1.reshape(NPAD)[:N]], axis=1)


# speedup vs baseline: 1.0896x; 1.0896x over previous
import functools

import jax
import jax.numpy as jnp
from jax import lax
from jax.experimental import pallas as pl
from jax.experimental.pallas import tpu as pltpu
from jax.experimental.pallas import tpu_sc as plsc

N = 100000
E = 6400000
NPAD = 100096
ROWS = NPAD // 128
NT = 32
EPT = E // NT
CHUNK = 2000
NCHUNK = EPT // CHUNK
SLICE = NPAD // 16

_MESH = dict(core_axis_name="c", subcore_axis_name="s")


def _worker(c, s):
    return s * 2 + c


def _fill(ref, n, value):
    vec = jnp.full((16,), value, jnp.float32)

    def _body(i, carry):
        ref[pl.ds(i * 16, 16)] = vec
        return carry

    lax.fori_loop(0, n // 16, _body, None)


@functools.partial(
    pl.kernel,
    out_type=jax.ShapeDtypeStruct((2 * NPAD,), jnp.float32),
    mesh=plsc.VectorSubcoreMesh(**_MESH),
    scratch_types=[
        pltpu.VMEM((CHUNK,), jnp.int32),
        pltpu.VMEM((CHUNK,), jnp.int32),
        pltpu.VMEM((CHUNK,), jnp.float32),
        pltpu.VMEM((SLICE,), jnp.float32),
        pltpu.VMEM_SHARED((NPAD,), jnp.float32),
        pltpu.SemaphoreType.DMA,
        pltpu.SemaphoreType.DMA,
        pltpu.SemaphoreType.DMA,
        pltpu.SemaphoreType.DMA,
    ],
)
def _sc_histogram(col_hbm, out_hbm, col0, col1, ones_v, stage_v, acc,
                  sl0, sl1, ss0, ss1):
    c = lax.axis_index("c")
    s = lax.axis_index("s")

    _fill(ones_v, CHUNK, 1.0)
    _fill(stage_v, SLICE, 0.0)
    sl = pl.ds(s * SLICE, SLICE)
    pltpu.sync_copy(stage_v, acc.at[sl])
    plsc.subcore_barrier()

    base = _worker(c, s) * EPT
    bufs = ((col0, sl0, ss0), (col1, sl1, ss1))

    def _src(ci):
        return col_hbm.at[pl.ds(base + ci * CHUNK, CHUNK)]

    def _ls(ci, b):
        pltpu.async_copy(_src(ci), bufs[b][0], bufs[b][1])

    def _lw(ci, b):
        pltpu.make_async_copy(_src(ci), bufs[b][0], bufs[b][1]).wait()

    def _scs(b):
        pltpu.async_copy(ones_v, acc.at[bufs[b][0]], bufs[b][2], add=True)

    def _scw(b):
        pltpu.make_async_copy(ones_v, acc.at[bufs[b][0]], bufs[b][2]).wait()

    _ls(0, 0)
    _ls(1, 1)

    def _body(k, carry):
        c0 = 2 * k
        _lw(c0, 0)
        _scs(0)
        _lw(c0 + 1, 1)
        _scs(1)
        _scw(0)
        _ls(c0 + 2, 0)
        _scw(1)
        _ls(c0 + 3, 1)
        return carry

    lax.fori_loop(0, NCHUNK // 2 - 1, _body, None)
    _lw(NCHUNK - 2, 0)
    _scs(0)
    _lw(NCHUNK - 1, 1)
    _scs(1)
    _scw(0)
    _scw(1)

    plsc.subcore_barrier()
    pltpu.sync_copy(acc.at[sl], stage_v)
    pltpu.sync_copy(stage_v, out_hbm.at[pl.ds(c * NPAD + s * SLICE, SLICE)])


def _make_edge_pass(packed, chunk):
    nchan = 2 if packed else 1
    nchunk = EPT // chunk
    assert EPT % chunk == 0 and nchunk % 2 == 0 and chunk % 16 == 0
    tab_dt = jnp.int32 if packed else jnp.float32
    out_sds = jax.ShapeDtypeStruct((nchan * 2 * NPAD,), jnp.float32)
    per_buf = [
        pltpu.VMEM((chunk,), jnp.int32),
        pltpu.VMEM((chunk,), jnp.int32),
    ] + [pltpu.VMEM((chunk,), jnp.float32)] * nchan + [
        pltpu.SemaphoreType.DMA,
        pltpu.SemaphoreType.DMA,
    ]
    scratch = per_buf + per_buf + [
        pltpu.VMEM((NPAD,), tab_dt),
    ] + [pltpu.VMEM_SHARED((NPAD,), jnp.float32)] * nchan

    pieces = []
    _off = 0
    while _off < SLICE:
        _sz = min(chunk, SLICE - _off)
        pieces.append((_off, _sz))
        _off += _sz
    assert all(sz > 0 and off % 8 == 0 and sz % 8 == 0 for off, sz in pieces)

    @functools.partial(
        pl.kernel,
        out_type=out_sds,
        mesh=plsc.VectorSubcoreMesh(**_MESH),
        scratch_types=scratch,
        compiler_params=pltpu.CompilerParams(needs_layout_passes=False),
    )
    def _pass(*args):
        tab_hbm, row_hbm, col_hbm, out_hbm = args[:4]
        rest = args[4:]
        nb = len(per_buf)
        bufs = (rest[:nb], rest[nb:2 * nb])
        table_v = rest[2 * nb]
        accs = rest[2 * nb + 1:]
        stage_v = bufs[0][2]

        c = lax.axis_index("c")
        s = lax.axis_index("s")

        pltpu.sync_copy(tab_hbm, table_v)
        _fill(stage_v, chunk, 0.0)
        for acc in accs:
            for off, sz in pieces:
                pltpu.sync_copy(stage_v.at[pl.ds(0, sz)],
                                acc.at[pl.ds(s * SLICE + off, sz)])
        plsc.subcore_barrier()

        base = _worker(c, s) * EPT

        def _rsrc(ci):
            return row_hbm.at[pl.ds(base + ci * chunk, chunk)]

        def _csrc(ci):
            return col_hbm.at[pl.ds(base + ci * chunk, chunk)]

        def _ls(ci, b):
            bb = bufs[b]
            pltpu.async_copy(_rsrc(ci), bb[0], bb[nchan + 2])
            pltpu.async_copy(_csrc(ci), bb[1], bb[nchan + 2])

        def _lw(ci, b):
            bb = bufs[b]
            pltpu.make_async_copy(_rsrc(ci), bb[0], bb[nchan + 2]).wait()
            pltpu.make_async_copy(_csrc(ci), bb[1], bb[nchan + 2]).wait()

        mask_hi = jnp.full((16,), -65536, jnp.int32)

        def _gather(b):
            bb = bufs[b]

            def _gbody(j, carry):
                dj = pl.ds(j * 16, 16)
                idx = bb[0][dj]
                w = plsc.load_gather(table_v, [idx])
                if packed:
                    bb[2][dj] = plsc.bitcast(
                        lax.shift_left(w, jnp.full((16,), 16, jnp.int32)),
                        jnp.float32)
                    bb[3][dj] = plsc.bitcast(lax.bitwise_and(w, mask_hi),
                                             jnp.float32)
                else:
                    bb[2][dj] = w
                return carry

            lax.fori_loop(0, chunk // 16, _gbody, None)

        def _scs(b):
            bb = bufs[b]
            for k in range(nchan):
                pltpu.async_copy(bb[2 + k], accs[k].at[bb[1]], bb[nchan + 3],
                                 add=True)

        def _scw(b):
            bb = bufs[b]
            for k in range(nchan):
                pltpu.make_async_copy(bb[2 + k], accs[k].at[bb[1]],
                                      bb[nchan + 3]).wait()

        _ls(0, 0)
        _ls(1, 1)

        def _body(k, carry):
            c0 = 2 * k
            _lw(c0, 0)
            _gather(0)
            _scs(0)
            _lw(c0 + 1, 1)
            _gather(1)
            _scs(1)
            _scw(0)
            _ls(c0 + 2, 0)
            _scw(1)
            _ls(c0 + 3, 1)
            return carry

        lax.fori_loop(0, nchunk // 2 - 1, _body, None)

        _lw(nchunk - 2, 0)
        _gather(0)
        _scs(0)
        _lw(nchunk - 1, 1)
        _gather(1)
        _scs(1)
        _scw(0)
        _scw(1)

        plsc.subcore_barrier()
        for k, acc in enumerate(accs):
            for off, sz in pieces:
                pltpu.sync_copy(acc.at[pl.ds(s * SLICE + off, sz)],
                                stage_v.at[pl.ds(0, sz)])
                pltpu.sync_copy(
                    stage_v.at[pl.ds(0, sz)],
                    out_hbm.at[pl.ds((2 * k + c) * NPAD + s * SLICE + off,
                                     sz)])

    return _pass


_sc_edge1 = _make_edge_pass(False, 4000)
_sc_edge2 = _make_edge_pass(True, 2000)


def _t1_body(cnt_ref, x_ref, dinv_ref, dis_ref, q_ref, xod_ref):
    deg = cnt_ref[0] + cnt_ref[1] + 1.0
    dis = lax.rsqrt(deg)
    dinv = 1.0 / deg
    x = x_ref[...]
    dinv_ref[...] = dinv
    dis_ref[...] = dis
    q_ref[...] = dis * x
    xod_ref[...] = x * dinv


def _t2_body(dis_ref, apart_ref, xod_ref, dinv_ref, w1_ref, b1_ref, w2_ref,
             rw_ref, pod0_ref, pod1_ref):
    dis = dis_ref[...]
    svec = dis * (apart_ref[0] + apart_ref[1]) + xod_ref[...]
    p0 = jnp.zeros_like(svec)
    p1 = jnp.zeros_like(svec)
    for k in range(16):
        hk = jnp.maximum(svec * w1_ref[0, k] + b1_ref[k], 0.0)
        p0 = p0 + hk * w2_ref[k, 0]
        p1 = p1 + hk * w2_ref[k, 1]
    dinv = dinv_ref[...]
    u0 = lax.bitcast_convert_type(
        lax.convert_element_type(dis * p0, jnp.bfloat16),
        jnp.uint16).astype(jnp.int32)
    u1 = lax.bitcast_convert_type(
        lax.convert_element_type(dis * p1, jnp.bfloat16),
        jnp.uint16).astype(jnp.int32)
    rw_ref[...] = jnp.bitwise_or(jnp.left_shift(u1, 16), u0)
    pod0_ref[...] = p0 * dinv
    pod1_ref[...] = p1 * dinv


def _t3_body(dis_ref, bc0_ref, bc1_ref, pod0_ref, pod1_ref, b2_ref,
             o0_ref, o1_ref):
    dis = dis_ref[...]
    t0 = dis * (bc0_ref[0] + bc0_ref[1]) + pod0_ref[...] + b2_ref[0]
    t1 = dis * (bc1_ref[0] + bc1_ref[1]) + pod1_ref[...] + b2_ref[1]
    m = jnp.maximum(t0, t1)
    e0 = jnp.exp(t0 - m)
    e1 = jnp.exp(t1 - m)
    den = e0 + e1
    o0_ref[...] = e0 / den
    o1_ref[...] = e1 / den


_f32 = jnp.float32
_blk = lambda: pl.BlockSpec(memory_space=pltpu.MemorySpace.VMEM)
_smem = lambda: pl.BlockSpec(memory_space=pltpu.MemorySpace.SMEM)
_VEC = jax.ShapeDtypeStruct((ROWS, 128), _f32)

_t1 = pl.pallas_call(
    _t1_body,
    out_shape=[_VEC, _VEC, _VEC, _VEC],
    in_specs=[_blk(), _blk()],
    out_specs=[_blk(), _blk(), _blk(), _blk()],
)

_VECI = jax.ShapeDtypeStruct((ROWS, 128), jnp.int32)
_t2 = pl.pallas_call(
    _t2_body,
    out_shape=[_VECI, _VEC, _VEC],
    in_specs=[_blk(), _blk(), _blk(), _blk(), _smem(), _smem(), _smem()],
    out_specs=[_blk(), _blk(), _blk()],
)

_t3 = pl.pallas_call(
    _t3_body,
    out_shape=[_VEC, _VEC],
    in_specs=[_blk(), _blk(), _blk(), _blk(), _blk(), _smem()],
    out_specs=[_blk(), _blk()],
)


def kernel(x, edge_index, W1, b1, W2, b2):
    row = edge_index[0]
    col = edge_index[1]
    xp = jnp.pad(x[:, 0], (0, NPAD - N)).reshape(ROWS, 128)

    cnt = _sc_histogram(col)
    dinv, dis, q, xod = _t1(cnt.reshape(2, ROWS, 128), xp)

    a_part = _sc_edge1(q.reshape(NPAD), row, col)
    rw, pod0, pod1 = _t2(dis, a_part.reshape(2, ROWS, 128), xod, dinv,
                         W1, b1, W2)

    b_part = _sc_edge2(rw.reshape(NPAD), row, col)
    b_part = b_part.reshape(2, 2, NPAD)
    bc0 = b_part[0].reshape(2, ROWS, 128)
    bc1 = b_part[1].reshape(2, ROWS, 128)
    o0, o1 = _t3(dis, bc0, bc1, pod0, pod1, b2)

    return jnp.stack([o0.reshape(NPAD)[:N], o1.reshape(NPAD)[:N]], axis=1)
